# baseline (speedup 1.0000x reference)
import numpy as np
import jax
import jax.numpy as jnp
from jax.experimental import pallas as pl
from jax.experimental.pallas import tpu as pltpu

ZERO = -100.0


def _sopa_kernel(x_ref, il_ref, w_ref, b_ref, eps_ref, es_ref, out_ref,
                 ts_ref, h_ref, sc_ref):
    Lc, B, D = x_ref.shape
    NP = 200
    N = es_ref.shape[1]
    S = NP - N
    Tb = 64
    l = pl.program_id(0)

    @pl.when(l == 0)
    def _init():
        lane = jax.lax.broadcasted_iota(jnp.int32, (B, NP), 1)
        h_ref[:, :] = jnp.where(lane < N, 0.0, ZERO)
        sc_ref[:, :] = jnp.full((B, N), ZERO, dtype=jnp.float32)

    xf = x_ref[:, :, :].reshape(Lc * B, D)
    ts_ref[:, :] = (
        jnp.dot(xf, w_ref[:, :], preferred_element_type=jnp.float32)
        + b_ref[:, :])

    def tile_body(bt, _):
        r0 = bt * Tb
        h = h_ref[pl.ds(r0, Tb), :]
        sc = sc_ref[pl.ds(r0, Tb), :]
        il = il_ref[pl.ds(r0, Tb), :]
        for j in range(Lc):
            ts = ts_ref[pl.ds(j * B + r0, Tb), :]
            tr0 = ts[:, :NP]
            tr1s = ts[:, NP:NP + S]
            shifted = jnp.concatenate(
                [jnp.full((Tb, N), ZERO, dtype=jnp.float32),
                 h[:, :S] + eps_ref[:, :]], axis=1)
            after = jnp.maximum(h, shifted)
            main = jnp.concatenate(
                [jnp.zeros((Tb, N), dtype=jnp.float32),
                 after[:, :S] + tr1s], axis=1)
            h = jnp.maximum(main, after + tr0)
            ev = h[:, 0:N]
            for p in range(1, NP // N):
                ev = jnp.where(es_ref[:, :] == p, h[:, p * N:(p + 1) * N], ev)
            act = il >= (l * Lc + j)
            sc = jnp.where(act, jnp.maximum(sc, ev), sc)
            out_ref[j, pl.ds(r0, Tb), :] = jnp.tanh(sc)
        h_ref[pl.ds(r0, Tb), :] = h
        sc_ref[pl.ds(r0, Tb), :] = sc
        return 0

    jax.lax.fori_loop(0, B // Tb, tile_body, 0)


def kernel(x, input_len, diags, bias, epsilon, end_states):
    L, B, D = x.shape
    N, Pm1 = epsilon.shape
    P = Pm1 + 1
    NP = N * P
    NC = NP + Pm1 * N

    n_i = np.arange(N)
    perm = np.empty(2 * NP, dtype=np.int32)
    for s in range(2):
        for p in range(P):
            perm[s * NP + p * N + n_i] = n_i * 2 * P + s * P + p
    perm = perm[:NC]
    w = jnp.transpose(diags[perm, :], (1, 0))
    b = bias[perm, 0][None, :]
    eps_row = jnp.transpose(epsilon, (1, 0)).reshape(1, Pm1 * N)
    es_row = end_states[:, 0][None, :].astype(jnp.int32)
    il = input_len.astype(jnp.int32)[:, None]

    Lc = 10
    grid = (L // Lc,)
    out = pl.pallas_call(
        _sopa_kernel,
        grid=grid,
        in_specs=[
            pl.BlockSpec((Lc, B, D), lambda l: (l, 0, 0)),
            pl.BlockSpec((B, 1), lambda l: (0, 0)),
            pl.BlockSpec((D, NC), lambda l: (0, 0)),
            pl.BlockSpec((1, NC), lambda l: (0, 0)),
            pl.BlockSpec((1, Pm1 * N), lambda l: (0, 0)),
            pl.BlockSpec((1, N), lambda l: (0, 0)),
        ],
        out_specs=pl.BlockSpec((Lc, B, N), lambda l: (l, 0, 0)),
        out_shape=jax.ShapeDtypeStruct((L, B, N), jnp.float32),
        scratch_shapes=[
            pltpu.VMEM((Lc * B, NC), jnp.float32),
            pltpu.VMEM((B, NP), jnp.float32),
            pltpu.VMEM((B, N), jnp.float32),
        ],
        compiler_params=pltpu.CompilerParams(
            dimension_semantics=("arbitrary",),
        ),
    )(x, il, w, b, eps_row, es_row)
    return out


# speedup vs baseline: 2.6624x; 2.6624x over previous
import numpy as np
import jax
import jax.numpy as jnp
from jax.experimental import pallas as pl
from jax.experimental.pallas import tpu as pltpu

ZERO = -100.0


def _sopa_kernel(x_ref, il_ref, w_ref, b_ref, eps_ref, es_ref, out_ref,
                 ts_ref, h_ref, sc_ref):
    Lc, B, D = x_ref.shape
    NC, NP = w_ref.shape[0], h_ref.shape[0]
    N = es_ref.shape[0]
    S = NP - N
    l = pl.program_id(0)

    @pl.when(l == 0)
    def _init():
        row = jax.lax.broadcasted_iota(jnp.int32, (NP, B), 0)
        h_ref[:, :] = jnp.where(row < N, 0.0, ZERO)
        sc_ref[:, :] = jnp.full((N, B), ZERO, dtype=jnp.float32)

    xf = x_ref[:, :, :].reshape(Lc * B, D)
    ts_ref[:, :] = jax.lax.dot_general(
        w_ref[:, :], xf, (((1,), (1,)), ((), ())),
        preferred_element_type=jnp.float32) + b_ref[:, :]

    def body(j, carry):
        h, sc = carry
        ts = ts_ref[:, pl.ds(j * B, B)]
        tr0 = ts[:NP, :]
        tr1s = ts[NP:, :]
        sh1h = jnp.concatenate(
            [jnp.full((N, B), ZERO, dtype=jnp.float32), h[:S, :]], axis=0)
        after = jnp.maximum(h, sh1h + eps_ref[:, :])
        main = jnp.concatenate(
            [jnp.zeros((N, B), dtype=jnp.float32),
             after[:S, :] + tr1s], axis=0)
        h = jnp.maximum(main, after + tr0)
        ev = h[:N, :]
        for p in range(1, NP // N):
            ev = jnp.where(es_ref[:, :] == p, h[p * N:(p + 1) * N, :], ev)
        act = il_ref[:, :] >= (l * Lc + j)
        sc = jnp.where(act, jnp.maximum(sc, ev), sc)
        out_ref[j] = jnp.transpose(jnp.tanh(sc), (1, 0))
        return h, sc

    h, sc = jax.lax.fori_loop(0, Lc, body, (h_ref[:, :], sc_ref[:, :]))
    h_ref[:, :] = h
    sc_ref[:, :] = sc


def kernel(x, input_len, diags, bias, epsilon, end_states):
    L, B, D = x.shape
    N, Pm1 = epsilon.shape
    P = Pm1 + 1
    NP = N * P
    NC = NP + Pm1 * N

    n_i = np.arange(N)
    perm = np.empty(2 * NP, dtype=np.int32)
    for s in range(2):
        for p in range(P):
            perm[s * NP + p * N + n_i] = n_i * 2 * P + s * P + p
    perm = perm[:NC]
    w = diags[perm, :]
    b = bias[perm, 0][:, None]
    eps_col = jnp.concatenate(
        [jnp.zeros((N,), jnp.float32),
         jnp.transpose(epsilon, (1, 0)).reshape(Pm1 * N)])[:, None]
    es_col = end_states[:, 0][:, None].astype(jnp.int32)
    il = input_len.astype(jnp.int32)[None, :]

    Lc = 10
    grid = (L // Lc,)
    out = pl.pallas_call(
        _sopa_kernel,
        grid=grid,
        in_specs=[
            pl.BlockSpec((Lc, B, D), lambda l: (l, 0, 0)),
            pl.BlockSpec((1, B), lambda l: (0, 0)),
            pl.BlockSpec((NC, D), lambda l: (0, 0)),
            pl.BlockSpec((NC, 1), lambda l: (0, 0)),
            pl.BlockSpec((NP, 1), lambda l: (0, 0)),
            pl.BlockSpec((N, 1), lambda l: (0, 0)),
        ],
        out_specs=pl.BlockSpec((Lc, B, N), lambda l: (l, 0, 0)),
        out_shape=jax.ShapeDtypeStruct((L, B, N), jnp.float32),
        scratch_shapes=[
            pltpu.VMEM((NC, Lc * B), jnp.float32),
            pltpu.VMEM((NP, B), jnp.float32),
            pltpu.VMEM((N, B), jnp.float32),
        ],
        compiler_params=pltpu.CompilerParams(
            dimension_semantics=("arbitrary",),
        ),
    )(x, il, w, b, eps_col, es_col)
    return out


# speedup vs baseline: 3.6884x; 1.3854x over previous
import numpy as np
import jax
import jax.numpy as jnp
from jax.experimental import pallas as pl
from jax.experimental.pallas import tpu as pltpu

ZERO = -100.0


def _sopa_kernel(x_ref, il_ref, w_ref, b_ref, eps_ref, es_ref, out_ref,
                 h_ref, sc_ref):
    Lc, B, D = x_ref.shape
    NC, NP = w_ref.shape[0], h_ref.shape[0]
    N = es_ref.shape[0]
    S = NP - N
    l = pl.program_id(0)

    @pl.when(l == 0)
    def _init():
        row = jax.lax.broadcasted_iota(jnp.int32, (NP, B), 0)
        h_ref[:, :] = jnp.where(row < N, 0.0, ZERO)
        sc_ref[:, :] = jnp.full((N, B), ZERO, dtype=jnp.float32)

    def trans_scores(j):
        return jax.lax.dot_general(
            w_ref[:, :], x_ref[j], (((1,), (1,)), ((), ())),
            preferred_element_type=jnp.float32) + b_ref[:, :]

    h = h_ref[:, :]
    sc = sc_ref[:, :]
    ts = trans_scores(0)
    for j in range(Lc):
        ts_next = trans_scores(j + 1) if j + 1 < Lc else None
        tr0 = ts[:NP, :]
        tr1s = ts[NP:, :]
        sh1h = jnp.concatenate(
            [jnp.full((N, B), ZERO, dtype=jnp.float32), h[:S, :]], axis=0)
        after = jnp.maximum(h, sh1h + eps_ref[:, :])
        main = jnp.concatenate(
            [jnp.zeros((N, B), dtype=jnp.float32),
             after[:S, :] + tr1s], axis=0)
        h = jnp.maximum(main, after + tr0)
        ev = h[:N, :]
        for p in range(1, NP // N):
            ev = jnp.where(es_ref[:, :] == p, h[p * N:(p + 1) * N, :], ev)
        act = il_ref[:, :] >= (l * Lc + j)
        sc = jnp.where(act, jnp.maximum(sc, ev), sc)
        out_ref[j] = jnp.transpose(jnp.tanh(sc), (1, 0))
        ts = ts_next
    h_ref[:, :] = h
    sc_ref[:, :] = sc


def kernel(x, input_len, diags, bias, epsilon, end_states):
    L, B, D = x.shape
    N, Pm1 = epsilon.shape
    P = Pm1 + 1
    NP = N * P
    NC = NP + Pm1 * N

    n_i = np.arange(N)
    perm = np.empty(2 * NP, dtype=np.int32)
    for s in range(2):
        for p in range(P):
            perm[s * NP + p * N + n_i] = n_i * 2 * P + s * P + p
    perm = perm[:NC]
    w = diags[perm, :]
    b = bias[perm, 0][:, None]
    eps_col = jnp.concatenate(
        [jnp.zeros((N,), jnp.float32),
         jnp.transpose(epsilon, (1, 0)).reshape(Pm1 * N)])[:, None]
    es_col = end_states[:, 0][:, None].astype(jnp.int32)
    il = input_len.astype(jnp.int32)[None, :]

    Lc = 10
    grid = (L // Lc,)
    out = pl.pallas_call(
        _sopa_kernel,
        grid=grid,
        in_specs=[
            pl.BlockSpec((Lc, B, D), lambda l: (l, 0, 0)),
            pl.BlockSpec((1, B), lambda l: (0, 0)),
            pl.BlockSpec((NC, D), lambda l: (0, 0)),
            pl.BlockSpec((NC, 1), lambda l: (0, 0)),
            pl.BlockSpec((NP, 1), lambda l: (0, 0)),
            pl.BlockSpec((N, 1), lambda l: (0, 0)),
        ],
        out_specs=pl.BlockSpec((Lc, B, N), lambda l: (l, 0, 0)),
        out_shape=jax.ShapeDtypeStruct((L, B, N), jnp.float32),
        scratch_shapes=[
            pltpu.VMEM((NP, B), jnp.float32),
            pltpu.VMEM((N, B), jnp.float32),
        ],
        compiler_params=pltpu.CompilerParams(
            dimension_semantics=("arbitrary",),
        ),
    )(x, il, w, b, eps_col, es_col)
    return out


# speedup vs baseline: 3.8125x; 1.0336x over previous
import numpy as np
import jax
import jax.numpy as jnp
from jax.experimental import pallas as pl
from jax.experimental.pallas import tpu as pltpu

ZERO = -100.0


def _sopa_kernel(x_ref, il_ref, w_ref, b_ref, eps_ref, es_ref, out_ref,
                 h_ref, sc_ref):
    Lc, B, D = x_ref.shape
    NC, NP = w_ref.shape[0], h_ref.shape[0]
    N = es_ref.shape[0]
    S = NP - N
    l = pl.program_id(0)

    @pl.when(l == 0)
    def _init():
        row = jax.lax.broadcasted_iota(jnp.int32, (NP, B), 0)
        h_ref[:, :] = jnp.where(row < N, 0.0, ZERO)
        sc_ref[:, :] = jnp.full((N, B), ZERO, dtype=jnp.float32)

    def trans_scores(j):
        return jax.lax.dot_general(
            w_ref[:, :], x_ref[j], (((1,), (1,)), ((), ())),
            preferred_element_type=jnp.float32) + b_ref[:, :]

    h = h_ref[:, :]
    sc = sc_ref[:, :]
    ts = trans_scores(0)
    eps_hi = eps_ref[N:, :]
    for j in range(Lc):
        ts_next = trans_scores(j + 1) if j + 1 < Lc else None
        tr0 = ts[:NP, :]
        tr1s = ts[NP:, :]
        after = jnp.concatenate(
            [h[:N, :], jnp.maximum(h[N:, :], h[:S, :] + eps_hi)], axis=0)
        sl = after + tr0
        h = jnp.concatenate(
            [jnp.maximum(sl[:N, :], 0.0),
             jnp.maximum(after[:S, :] + tr1s, sl[N:, :])], axis=0)
        ev = h[:N, :]
        for p in range(1, NP // N):
            ev = jnp.where(es_ref[:, :] == p, h[p * N:(p + 1) * N, :], ev)
        pen = jnp.where(il_ref[:, :] >= (l * Lc + j), 0.0, -3e8)
        sc = jnp.maximum(sc, ev + pen)
        out_ref[j] = jnp.transpose(jnp.tanh(sc), (1, 0))
        ts = ts_next
    h_ref[:, :] = h
    sc_ref[:, :] = sc


def kernel(x, input_len, diags, bias, epsilon, end_states):
    L, B, D = x.shape
    N, Pm1 = epsilon.shape
    P = Pm1 + 1
    NP = N * P
    NC = NP + Pm1 * N

    n_i = np.arange(N)
    perm = np.empty(2 * NP, dtype=np.int32)
    for s in range(2):
        for p in range(P):
            perm[s * NP + p * N + n_i] = n_i * 2 * P + s * P + p
    perm = perm[:NC]
    w = diags[perm, :]
    b = bias[perm, 0][:, None]
    eps_col = jnp.concatenate(
        [jnp.zeros((N,), jnp.float32),
         jnp.transpose(epsilon, (1, 0)).reshape(Pm1 * N)])[:, None]
    es_col = end_states[:, 0][:, None].astype(jnp.int32)
    il = input_len.astype(jnp.int32)[None, :]

    Lc = 10
    grid = (L // Lc,)
    out = pl.pallas_call(
        _sopa_kernel,
        grid=grid,
        in_specs=[
            pl.BlockSpec((Lc, B, D), lambda l: (l, 0, 0)),
            pl.BlockSpec((1, B), lambda l: (0, 0)),
            pl.BlockSpec((NC, D), lambda l: (0, 0)),
            pl.BlockSpec((NC, 1), lambda l: (0, 0)),
            pl.BlockSpec((NP, 1), lambda l: (0, 0)),
            pl.BlockSpec((N, 1), lambda l: (0, 0)),
        ],
        out_specs=pl.BlockSpec((Lc, B, N), lambda l: (l, 0, 0)),
        out_shape=jax.ShapeDtypeStruct((L, B, N), jnp.float32),
        scratch_shapes=[
            pltpu.VMEM((NP, B), jnp.float32),
            pltpu.VMEM((N, B), jnp.float32),
        ],
        compiler_params=pltpu.CompilerParams(
            dimension_semantics=("arbitrary",),
        ),
    )(x, il, w, b, eps_col, es_col)
    return out
